# baseline (speedup 1.0000x reference)
import jax
import jax.numpy as jnp
from jax import lax
from jax.experimental import pallas as pl
from jax.experimental.pallas import tpu as pltpu
from jax.experimental.pallas import tpu_sc as plsc

N = 10000
E_RAW = 320000
E_TOT = E_RAW + N
D = 128
OC = 64
PEXT = 72
NCORE = 2
NSUB = 16
NW = NCORE * NSUB
BLK = 128
NBLK = 84
EPW = NBLK * BLK
E_PAD = NW * EPW
N_PAD = 10240
TROWS = N_PAD // NSUB
GRIDPTS = (-2.0, -2.0 / 3.0, 2.0 / 3.0, 2.0)
INV_DENOM = 0.75
R = 200
NBR = N // R
NGRAPH = 64
NCLS = 16


def _fastkan(x, lng, lnb, wspl, wbt, bb):
    mu = jnp.mean(x, axis=1, keepdims=True)
    var = jnp.mean((x - mu) ** 2, axis=1, keepdims=True)
    xn = (x - mu) * lax.rsqrt(var + 1e-5) * lng + lnb
    h = jnp.dot(jax.nn.silu(x), wbt, preferred_element_type=jnp.float32) + bb
    for g in range(4):
        basis = jnp.exp(-(((xn - GRIDPTS[g]) * INV_DENOM) ** 2))
        h = h + jnp.dot(basis, wspl[g], preferred_element_type=jnp.float32)
    return h


def _att_out(acc4, bias):
    s = acc4[0] + acc4[1]
    den0 = s[0, :, OC:OC + 1] + 1e-16
    den1 = s[1, :, OC:OC + 1] + 1e-16
    x = jnp.concatenate([s[0, :, :OC] / den0, s[1, :, :OC] / den1],
                        axis=1) + bias
    return jax.nn.silu(x)


def _emit_ext(h, swa, swb, ha_ref, hb_ref, tab_ref):
    rows = h.shape[0]
    asc = jnp.dot(h, swa, preferred_element_type=jnp.float32)
    adt = jnp.dot(h, swb, preferred_element_type=jnp.float32)
    z1 = jnp.zeros((rows, 1), jnp.float32)
    z6 = jnp.zeros((rows, 6), jnp.float32)
    ha_ref[...] = jnp.concatenate([h[:, :OC], z1, asc[:, 0:1], z6], axis=1)
    hb_ref[...] = jnp.concatenate([h[:, OC:], z1, asc[:, 1:2], z6], axis=1)
    tab_ref[...] = jnp.concatenate([adt, z6], axis=1)


def _tc_in_body(x_ref, lng_ref, lnb_ref, wspl_ref, wbt_ref, bb_ref, swa_ref,
                swb_ref, ha_ref, hb_ref, tab_ref):
    h = _fastkan(x_ref[...], lng_ref[...], lnb_ref[...], wspl_ref[...],
                 wbt_ref[...], bb_ref[...])
    _emit_ext(h, swa_ref[...], swb_ref[...], ha_ref, hb_ref, tab_ref)


def _tc_mid_body(acc_ref, bias_ref, lng_ref, lnb_ref, wspl_ref, wbt_ref,
                 bb_ref, swa_ref, swb_ref, ha_ref, hb_ref, tab_ref):
    x = _att_out(acc_ref[...], bias_ref[...])
    h = _fastkan(x, lng_ref[...], lnb_ref[...], wspl_ref[...], wbt_ref[...],
                 bb_ref[...])
    _emit_ext(h, swa_ref[...], swb_ref[...], ha_ref, hb_ref, tab_ref)


def _tc_pool_body(acc_ref, bias_ref, batch_ref, pooled_ref):
    x = _att_out(acc_ref[...], bias_ref[...])
    b = batch_ref[...].reshape(1, R)
    ids = lax.broadcasted_iota(jnp.int32, (NGRAPH, R), 0)
    oh = jnp.where(ids == b, 1.0, 0.0)
    contrib = jnp.dot(oh, x, preferred_element_type=jnp.float32)

    @pl.when(pl.program_id(0) == 0)
    def _():
        pooled_ref[...] = contrib

    @pl.when(pl.program_id(0) != 0)
    def _():
        pooled_ref[...] = pooled_ref[...] + contrib


def _tc_head_body(p_ref, lng_ref, lnb_ref, wspl_ref, wbt_ref, bb_ref,
                  out_ref):
    h = _fastkan(p_ref[...], lng_ref[...], lnb_ref[...], wspl_ref[...],
                 wbt_ref[...], bb_ref[...])
    m = jnp.max(h, axis=1, keepdims=True)
    lse = jnp.log(jnp.sum(jnp.exp(h - m), axis=1, keepdims=True)) + m
    out_ref[...] = h - lse


def _full(shape):
    nd = len(shape)
    return pl.BlockSpec(shape, lambda j: (0,) * nd)


_ACC_SPEC = pl.BlockSpec((2, 2, R, PEXT), lambda j: (0, 0, j, 0))
_EXT_OUT = [
    pl.BlockSpec((R, PEXT), lambda j: (j, 0)),
    pl.BlockSpec((R, PEXT), lambda j: (j, 0)),
    pl.BlockSpec((R, 8), lambda j: (j, 0)),
]
_EXT_SHAPE = [
    jax.ShapeDtypeStruct((N, PEXT), jnp.float32),
    jax.ShapeDtypeStruct((N, PEXT), jnp.float32),
    jax.ShapeDtypeStruct((N, 8), jnp.float32),
]


def _tc_transform(x, lng, lnb, wspl, wbt, bb, swa, swb):
    return pl.pallas_call(
        _tc_in_body,
        grid=(NBR,),
        in_specs=[
            pl.BlockSpec((R, D), lambda j: (j, 0)),
            _full((1, D)), _full((1, D)), _full((4, D, D)),
            _full((D, D)), _full((1, D)), _full((D, 2)), _full((D, 2)),
        ],
        out_specs=_EXT_OUT,
        out_shape=_EXT_SHAPE,
    )(x, lng, lnb, wspl, wbt, bb, swa, swb)


def _tc_mid(accb, bias, lng, lnb, wspl, wbt, bb, swa, swb):
    return pl.pallas_call(
        _tc_mid_body,
        grid=(NBR,),
        in_specs=[
            _ACC_SPEC,
            _full((1, D)), _full((1, D)), _full((1, D)), _full((4, D, D)),
            _full((D, D)), _full((1, D)), _full((D, 2)), _full((D, 2)),
        ],
        out_specs=_EXT_OUT,
        out_shape=_EXT_SHAPE,
    )(accb, bias, lng, lnb, wspl, wbt, bb, swa, swb)


def _tc_pool(accb, bias, batch3):
    return pl.pallas_call(
        _tc_pool_body,
        grid=(NBR,),
        in_specs=[
            _ACC_SPEC,
            _full((1, D)),
            pl.BlockSpec((1, 1, R), lambda j: (j, 0, 0)),
        ],
        out_specs=pl.BlockSpec((NGRAPH, D), lambda j: (0, 0)),
        out_shape=jax.ShapeDtypeStruct((NGRAPH, D), jnp.float32),
    )(accb, bias, batch3)


def _tc_head(pooled, lng, lnb, wspl, wbt, bb):
    return pl.pallas_call(
        _tc_head_body,
        grid=(1,),
        in_specs=[
            _full((NGRAPH, D)), _full((1, D)), _full((1, D)),
            _full((4, D, NCLS)), _full((D, NCLS)), _full((1, NCLS)),
        ],
        out_specs=_full((NGRAPH, NCLS)),
        out_shape=jax.ShapeDtypeStruct((NGRAPH, NCLS), jnp.float32),
    )(pooled, lng, lnb, wspl, wbt, bb)


def _sc_body(ha, hb, tab, em, out,
             idx_v, rows_v, drows_v, wb_v, acc,
             gs0, gs1, gs2, ss0, ss1, ss2):
    c = lax.axis_index("c")
    s = lax.axis_index("s")
    wid = c * NSUB + s
    base = wid * EPW
    rslice = pl.ds(s * TROWS, TROWS)
    gsems = (gs0, gs1, gs2)
    ssems = (ss0, ss1, ss2)

    z16 = jnp.zeros((16,), jnp.int32)
    zf16 = jnp.zeros((16,), jnp.float32)
    lane = lax.iota(jnp.int32, 16)

    for p, href in enumerate((ha, hb)):

        def issue(j, b):
            pltpu.sync_copy(em.at[wid, j], idx_v.at[b])
            pltpu.async_copy(href.at[idx_v.at[b, 0]], rows_v.at[b],
                             gsems[b])
            pltpu.async_copy(tab.at[idx_v.at[b, 1]], drows_v.at[b],
                             gsems[b])

        def wait_gathers(b):
            pltpu.make_async_copy(href.at[idx_v.at[b, 0]], rows_v.at[b],
                                  gsems[b]).wait()
            pltpu.make_async_copy(tab.at[idx_v.at[b, 1]], drows_v.at[b],
                                  gsems[b]).wait()

        def wait_scatter(b):
            pltpu.make_async_copy(rows_v.at[b], acc.at[idx_v.at[b, 1]],
                                  ssems[b]).wait()

        @plsc.parallel_loop(0, BLK, unroll=8)
        def _(i):
            for k in range(4):
                rows_v[0, i, pl.ds(k * 16, 16)] = zf16
            plsc.store_scatter(rows_v.at[0], [z16 + i, OC + lane], zf16,
                               mask=lane < PEXT - OC)

        for t in range(TROWS // BLK):
            pltpu.sync_copy(rows_v.at[0],
                            acc.at[pl.ds(s * TROWS + t * BLK, BLK)])
        issue(0, 0)
        issue(1, 1)
        plsc.subcore_barrier()

        def grp(jo, carry):
            for q in (0, 1, 2):
                b = q
                j = jo * 3 + q
                wait_gathers(b)
                for g in range(BLK // 16):
                    sl = pl.ds(g * 16, 16)
                    eidx = g * 16 + lane
                    a_s = plsc.load_gather(rows_v.at[b],
                                           [eidx, z16 + OC + 1])
                    a_d = plsc.load_gather(drows_v.at[b], [eidx, z16 + p])
                    e = a_s + a_d
                    e = jnp.where(e >= 0.0, e, 0.2 * e)
                    eid = base + j * BLK + eidx
                    wb_v[0, sl] = jnp.where(eid < E_TOT, jnp.exp(e), 0.0)

                @plsc.parallel_loop(0, BLK, unroll=8)
                def _(i):
                    ii = z16 + i
                    w = plsc.load_gather(wb_v, [z16, ii])
                    for k in range(4):
                        ksl = pl.ds(k * 16, 16)
                        rows_v[b, i, ksl] = rows_v[b, i, ksl] * w
                    tailv = jnp.where(lane == 0, w, 0.0)
                    plsc.store_scatter(rows_v.at[b], [ii, OC + lane], tailv,
                                       mask=lane < PEXT - OC)

                pltpu.async_copy(rows_v.at[b], acc.at[idx_v.at[b, 1]],
                                 ssems[b], add=True)
                bn = (q + 2) % 3
                jn = j + 2

                @pl.when(jn < NBLK)
                def _():
                    @pl.when(j >= 1)
                    def _():
                        wait_scatter(bn)

                    issue(jn, bn)
            return carry

        lax.fori_loop(0, NBLK // 3, grp, 0)
        for b in ((NBLK - 3) % 3, (NBLK - 2) % 3, (NBLK - 1) % 3):
            wait_scatter(b)
        plsc.subcore_barrier()
        pltpu.sync_copy(acc.at[rslice], out.at[c, p, rslice])


_sc_aggregate = pl.kernel(
    _sc_body,
    out_type=jax.ShapeDtypeStruct((NCORE, 2, N_PAD, PEXT), jnp.float32),
    mesh=plsc.VectorSubcoreMesh(core_axis_name="c", subcore_axis_name="s"),
    compiler_params=pltpu.CompilerParams(
        use_tc_tiling_on_sc=False, needs_layout_passes=False),
    scratch_types=[
        pltpu.VMEM((3, 2, BLK), jnp.int32),
        pltpu.VMEM((3, BLK, PEXT), jnp.float32),
        pltpu.VMEM((3, BLK, 8), jnp.float32),
        pltpu.VMEM((1, BLK), jnp.float32),
        pltpu.VMEM_SHARED((N_PAD, PEXT), jnp.float32),
        pltpu.SemaphoreType.DMA,
        pltpu.SemaphoreType.DMA,
        pltpu.SemaphoreType.DMA,
        pltpu.SemaphoreType.DMA,
        pltpu.SemaphoreType.DMA,
        pltpu.SemaphoreType.DMA,
    ],
)


def _score_mats(asrc, adst):
    swa = jnp.zeros((D, 2), jnp.float32)
    swa = swa.at[:OC, 0].set(asrc[0])
    swa = swa.at[OC:, 1].set(asrc[1])
    swb = jnp.zeros((D, 2), jnp.float32)
    swb = swb.at[:OC, 0].set(adst[0])
    swb = swb.at[OC:, 1].set(adst[1])
    return swa, swb


def kernel(x, edge_index, batch, ln_g0, ln_b0, Ws0, Wb0, bb0, asrc0, adst0,
           bias0, ln_g1, ln_b1, Ws1, Wb1, bb1, asrc1, adst1, bias1,
           ln_gr, ln_br, Wsr, Wbr, bbr):
    loops = jnp.arange(N, dtype=edge_index.dtype)
    pad = jnp.zeros((E_PAD - E_TOT,), edge_index.dtype)
    src3 = jnp.concatenate([edge_index[0], loops, pad]).astype(jnp.int32)
    dst3 = jnp.concatenate([edge_index[1], loops, pad]).astype(jnp.int32)
    em = jnp.stack([src3.reshape(NW, NBLK, BLK),
                    dst3.reshape(NW, NBLK, BLK)], axis=2)
    batch3 = batch.astype(jnp.int32).reshape(NBR, 1, R)

    def prep(Ws, out_ch):
        return Ws.reshape(out_ch, D, 4).transpose(2, 1, 0)

    lng0, lnb0 = ln_g0.reshape(1, D), ln_b0.reshape(1, D)
    lng1, lnb1 = ln_g1.reshape(1, D), ln_b1.reshape(1, D)
    lngr, lnbr = ln_gr.reshape(1, D), ln_br.reshape(1, D)
    swa0, swb0 = _score_mats(asrc0, adst0)
    swa1, swb1 = _score_mats(asrc1, adst1)

    ha, hb, tab = _tc_transform(x, lng0, lnb0, prep(Ws0, D), Wb0.T,
                                bb0.reshape(1, D), swa0, swb0)
    accb = _sc_aggregate(ha, hb, tab, em)
    ha, hb, tab = _tc_mid(accb, bias0.reshape(1, D), lng1, lnb1,
                          prep(Ws1, D), Wb1.T, bb1.reshape(1, D),
                          swa1, swb1)
    accb = _sc_aggregate(ha, hb, tab, em)
    pooled = _tc_pool(accb, bias1.reshape(1, D), batch3)
    return _tc_head(pooled, lngr, lnbr, prep(Wsr, NCLS), Wbr.T,
                    bbr.reshape(1, NCLS))


# speedup vs baseline: 1.4881x; 1.4881x over previous
import jax
import jax.numpy as jnp
from jax import lax
from jax.experimental import pallas as pl
from jax.experimental.pallas import tpu as pltpu
from jax.experimental.pallas import tpu_sc as plsc

N = 10000
E_RAW = 320000
E_TOT = E_RAW + N
D = 128
OC = 64
PEXT = 72
NCORE = 2
NSUB = 16
NW = NCORE * NSUB
BLK = 128
NBLK = 82
EPW = NBLK * BLK
E_PAD = NW * EPW
N_PAD = 10240
TROWS = N_PAD // NSUB
GRIDPTS = (-2.0, -2.0 / 3.0, 2.0 / 3.0, 2.0)
INV_DENOM = 0.75
R = 200
NBR = N // R
NGRAPH = 64
NCLS = 16


def _fastkan(x, lng, lnb, wspl, wbt, bb):
    mu = jnp.mean(x, axis=1, keepdims=True)
    var = jnp.mean((x - mu) ** 2, axis=1, keepdims=True)
    xn = (x - mu) * lax.rsqrt(var + 1e-5) * lng + lnb
    h = jnp.dot(jax.nn.silu(x), wbt, preferred_element_type=jnp.float32) + bb
    for g in range(4):
        basis = jnp.exp(-(((xn - GRIDPTS[g]) * INV_DENOM) ** 2))
        h = h + jnp.dot(basis, wspl[g], preferred_element_type=jnp.float32)
    return h


def _att_out(acc4, bias):
    s = acc4[0] + acc4[1]
    den0 = s[0, :, OC:OC + 1] + 1e-16
    den1 = s[1, :, OC:OC + 1] + 1e-16
    x = jnp.concatenate([s[0, :, :OC] / den0, s[1, :, :OC] / den1],
                        axis=1) + bias
    return jax.nn.silu(x)


def _emit_ext(h, swa, swb, ha_ref, hb_ref, tab_ref):
    rows = h.shape[0]
    asc = jnp.dot(h, swa, preferred_element_type=jnp.float32)
    adt = jnp.dot(h, swb, preferred_element_type=jnp.float32)
    z1 = jnp.zeros((rows, 1), jnp.float32)
    z6 = jnp.zeros((rows, 6), jnp.float32)
    ha_ref[...] = jnp.concatenate([h[:, :OC], z1, asc[:, 0:1], z6], axis=1)
    hb_ref[...] = jnp.concatenate([h[:, OC:], z1, asc[:, 1:2], z6], axis=1)
    tab_ref[...] = jnp.concatenate([adt, z6], axis=1)


def _tc_in_body(x_ref, lng_ref, lnb_ref, wspl_ref, wbt_ref, bb_ref, swa_ref,
                swb_ref, ha_ref, hb_ref, tab_ref):
    h = _fastkan(x_ref[...], lng_ref[...], lnb_ref[...], wspl_ref[...],
                 wbt_ref[...], bb_ref[...])
    _emit_ext(h, swa_ref[...], swb_ref[...], ha_ref, hb_ref, tab_ref)


def _tc_mid_body(acc_ref, bias_ref, lng_ref, lnb_ref, wspl_ref, wbt_ref,
                 bb_ref, swa_ref, swb_ref, ha_ref, hb_ref, tab_ref):
    x = _att_out(acc_ref[...], bias_ref[...])
    h = _fastkan(x, lng_ref[...], lnb_ref[...], wspl_ref[...], wbt_ref[...],
                 bb_ref[...])
    _emit_ext(h, swa_ref[...], swb_ref[...], ha_ref, hb_ref, tab_ref)


def _tc_pool_body(acc_ref, bias_ref, batch_ref, pooled_ref):
    x = _att_out(acc_ref[...], bias_ref[...])
    b = batch_ref[...].reshape(1, R)
    ids = lax.broadcasted_iota(jnp.int32, (NGRAPH, R), 0)
    oh = jnp.where(ids == b, 1.0, 0.0)
    contrib = jnp.dot(oh, x, preferred_element_type=jnp.float32)

    @pl.when(pl.program_id(0) == 0)
    def _():
        pooled_ref[...] = contrib

    @pl.when(pl.program_id(0) != 0)
    def _():
        pooled_ref[...] = pooled_ref[...] + contrib


def _tc_head_body(p_ref, lng_ref, lnb_ref, wspl_ref, wbt_ref, bb_ref,
                  out_ref):
    h = _fastkan(p_ref[...], lng_ref[...], lnb_ref[...], wspl_ref[...],
                 wbt_ref[...], bb_ref[...])
    m = jnp.max(h, axis=1, keepdims=True)
    lse = jnp.log(jnp.sum(jnp.exp(h - m), axis=1, keepdims=True)) + m
    out_ref[...] = h - lse


def _full(shape):
    nd = len(shape)
    return pl.BlockSpec(shape, lambda j: (0,) * nd)


_ACC_SPEC = pl.BlockSpec((2, 2, R, PEXT), lambda j: (0, 0, j, 0))
_EXT_OUT = [
    pl.BlockSpec((R, PEXT), lambda j: (j, 0)),
    pl.BlockSpec((R, PEXT), lambda j: (j, 0)),
    pl.BlockSpec((R, 8), lambda j: (j, 0)),
]
_EXT_SHAPE = [
    jax.ShapeDtypeStruct((N, PEXT), jnp.float32),
    jax.ShapeDtypeStruct((N, PEXT), jnp.float32),
    jax.ShapeDtypeStruct((N, 8), jnp.float32),
]


def _tc_transform(x, lng, lnb, wspl, wbt, bb, swa, swb):
    return pl.pallas_call(
        _tc_in_body,
        grid=(NBR,),
        in_specs=[
            pl.BlockSpec((R, D), lambda j: (j, 0)),
            _full((1, D)), _full((1, D)), _full((4, D, D)),
            _full((D, D)), _full((1, D)), _full((D, 2)), _full((D, 2)),
        ],
        out_specs=_EXT_OUT,
        out_shape=_EXT_SHAPE,
    )(x, lng, lnb, wspl, wbt, bb, swa, swb)


def _tc_mid(accb, bias, lng, lnb, wspl, wbt, bb, swa, swb):
    return pl.pallas_call(
        _tc_mid_body,
        grid=(NBR,),
        in_specs=[
            _ACC_SPEC,
            _full((1, D)), _full((1, D)), _full((1, D)), _full((4, D, D)),
            _full((D, D)), _full((1, D)), _full((D, 2)), _full((D, 2)),
        ],
        out_specs=_EXT_OUT,
        out_shape=_EXT_SHAPE,
    )(accb, bias, lng, lnb, wspl, wbt, bb, swa, swb)


def _tc_pool(accb, bias, batch3):
    return pl.pallas_call(
        _tc_pool_body,
        grid=(NBR,),
        in_specs=[
            _ACC_SPEC,
            _full((1, D)),
            pl.BlockSpec((1, 1, R), lambda j: (j, 0, 0)),
        ],
        out_specs=pl.BlockSpec((NGRAPH, D), lambda j: (0, 0)),
        out_shape=jax.ShapeDtypeStruct((NGRAPH, D), jnp.float32),
    )(accb, bias, batch3)


def _tc_head(pooled, lng, lnb, wspl, wbt, bb):
    return pl.pallas_call(
        _tc_head_body,
        grid=(1,),
        in_specs=[
            _full((NGRAPH, D)), _full((1, D)), _full((1, D)),
            _full((4, D, NCLS)), _full((D, NCLS)), _full((1, NCLS)),
        ],
        out_specs=_full((NGRAPH, NCLS)),
        out_shape=jax.ShapeDtypeStruct((NGRAPH, NCLS), jnp.float32),
    )(pooled, lng, lnb, wspl, wbt, bb)


def _sc_body(ha, hb, tab, srcm, dstm, out,
             sidx_v, didx_v, rows_v, drows_v, wb_v, acc, sem0, sem1):
    c = lax.axis_index("c")
    s = lax.axis_index("s")
    wid = c * NSUB + s
    base = wid * EPW
    rslice = pl.ds(s * TROWS, TROWS)
    sems = (sem0, sem1)

    z16 = jnp.zeros((16,), jnp.int32)
    zf16 = jnp.zeros((16,), jnp.float32)
    lane = lax.iota(jnp.int32, 16)

    for p, href in enumerate((ha, hb)):

        def issue(j, b):
            pltpu.sync_copy(srcm.at[wid, pl.ds(j, 1)], sidx_v.at[b])
            pltpu.sync_copy(dstm.at[wid, pl.ds(j, 1)], didx_v.at[b])
            pltpu.async_copy(href.at[sidx_v.at[b, 0]], rows_v.at[b], sems[b])
            pltpu.async_copy(tab.at[didx_v.at[b, 0]], drows_v.at[b], sems[b])

        def wait_gathers(b):
            pltpu.make_async_copy(href.at[sidx_v.at[b, 0]], rows_v.at[b],
                                  sems[b]).wait()
            pltpu.make_async_copy(tab.at[didx_v.at[b, 0]], drows_v.at[b],
                                  sems[b]).wait()

        @plsc.parallel_loop(0, BLK, unroll=8)
        def _(i):
            for k in range(4):
                rows_v[0, i, pl.ds(k * 16, 16)] = zf16
            plsc.store_scatter(rows_v.at[0], [z16 + i, OC + lane], zf16,
                               mask=lane < PEXT - OC)

        for t in range(TROWS // BLK):
            pltpu.sync_copy(rows_v.at[0],
                            acc.at[pl.ds(s * TROWS + t * BLK, BLK)])
        issue(0, 0)
        issue(1, 1)
        plsc.subcore_barrier()

        def grp(jo, carry):
            for b in (0, 1):
                j = jo * 2 + b
                wait_gathers(b)
                for g in range(BLK // 16):
                    sl = pl.ds(g * 16, 16)
                    eidx = g * 16 + lane
                    a_s = plsc.load_gather(rows_v.at[b],
                                           [eidx, z16 + OC + 1])
                    a_d = plsc.load_gather(drows_v.at[b], [eidx, z16 + p])
                    e = a_s + a_d
                    e = jnp.where(e >= 0.0, e, 0.2 * e)
                    eid = base + j * BLK + eidx
                    wb_v[0, sl] = jnp.where(eid < E_TOT, jnp.exp(e), 0.0)

                @plsc.parallel_loop(0, BLK, unroll=8)
                def _(i):
                    ii = z16 + i
                    w = plsc.load_gather(wb_v, [z16, ii])
                    for k in range(4):
                        ksl = pl.ds(k * 16, 16)
                        rows_v[b, i, ksl] = rows_v[b, i, ksl] * w
                    tailv = jnp.where(lane == 0, w, 0.0)
                    plsc.store_scatter(rows_v.at[b], [ii, OC + lane], tailv,
                                       mask=lane < PEXT - OC)

                pltpu.sync_copy(rows_v.at[b], acc.at[didx_v.at[b, 0]],
                                add=True)
                jn = j + 2

                @pl.when(jn < NBLK)
                def _():
                    issue(jn, b)
            return carry

        lax.fori_loop(0, NBLK // 2, grp, 0)
        plsc.subcore_barrier()
        pltpu.sync_copy(acc.at[rslice], out.at[c, p, rslice])


_sc_aggregate = pl.kernel(
    _sc_body,
    out_type=jax.ShapeDtypeStruct((NCORE, 2, N_PAD, PEXT), jnp.float32),
    mesh=plsc.VectorSubcoreMesh(core_axis_name="c", subcore_axis_name="s"),
    compiler_params=pltpu.CompilerParams(
        use_tc_tiling_on_sc=False, needs_layout_passes=False),
    scratch_types=[
        pltpu.VMEM((2, 1, BLK), jnp.int32),
        pltpu.VMEM((2, 1, BLK), jnp.int32),
        pltpu.VMEM((2, BLK, PEXT), jnp.float32),
        pltpu.VMEM((2, BLK, 8), jnp.float32),
        pltpu.VMEM((1, BLK), jnp.float32),
        pltpu.VMEM_SHARED((N_PAD, PEXT), jnp.float32),
        pltpu.SemaphoreType.DMA,
        pltpu.SemaphoreType.DMA,
    ],
)


def _score_mats(asrc, adst):
    swa = jnp.zeros((D, 2), jnp.float32)
    swa = swa.at[:OC, 0].set(asrc[0])
    swa = swa.at[OC:, 1].set(asrc[1])
    swb = jnp.zeros((D, 2), jnp.float32)
    swb = swb.at[:OC, 0].set(adst[0])
    swb = swb.at[OC:, 1].set(adst[1])
    return swa, swb


def kernel(x, edge_index, batch, ln_g0, ln_b0, Ws0, Wb0, bb0, asrc0, adst0,
           bias0, ln_g1, ln_b1, Ws1, Wb1, bb1, asrc1, adst1, bias1,
           ln_gr, ln_br, Wsr, Wbr, bbr):
    loops = jnp.arange(N, dtype=edge_index.dtype)
    pad = jnp.zeros((E_PAD - E_TOT,), edge_index.dtype)
    src3 = jnp.concatenate([edge_index[0], loops, pad]).astype(jnp.int32)
    dst3 = jnp.concatenate([edge_index[1], loops, pad]).astype(jnp.int32)
    src3 = src3.reshape(NW, NBLK, BLK)
    dst3 = dst3.reshape(NW, NBLK, BLK)
    batch3 = batch.astype(jnp.int32).reshape(NBR, 1, R)

    def prep(Ws, out_ch):
        return Ws.reshape(out_ch, D, 4).transpose(2, 1, 0)

    lng0, lnb0 = ln_g0.reshape(1, D), ln_b0.reshape(1, D)
    lng1, lnb1 = ln_g1.reshape(1, D), ln_b1.reshape(1, D)
    lngr, lnbr = ln_gr.reshape(1, D), ln_br.reshape(1, D)
    swa0, swb0 = _score_mats(asrc0, adst0)
    swa1, swb1 = _score_mats(asrc1, adst1)

    ha, hb, tab = _tc_transform(x, lng0, lnb0, prep(Ws0, D), Wb0.T,
                                bb0.reshape(1, D), swa0, swb0)
    accb = _sc_aggregate(ha, hb, tab, src3, dst3)
    ha, hb, tab = _tc_mid(accb, bias0.reshape(1, D), lng1, lnb1,
                          prep(Ws1, D), Wb1.T, bb1.reshape(1, D),
                          swa1, swb1)
    accb = _sc_aggregate(ha, hb, tab, src3, dst3)
    pooled = _tc_pool(accb, bias1.reshape(1, D), batch3)
    return _tc_head(pooled, lngr, lnbr, prep(Wsr, NCLS), Wbr.T,
                    bbr.reshape(1, NCLS))


# speedup vs baseline: 2.3726x; 1.5944x over previous
import jax
import jax.numpy as jnp
from jax import lax
from jax.experimental import pallas as pl
from jax.experimental.pallas import tpu as pltpu
from jax.experimental.pallas import tpu_sc as plsc

N = 10000
E_RAW = 320000
E_TOT = E_RAW + N
D = 128
OC = 64
PEXT = 72
NCORE = 2
NSUB = 16
NW = NCORE * NSUB
BLK = 128
NBLK = 82
EPW = NBLK * BLK
E_PAD = NW * EPW
N_PAD = 10240
TROWS = N_PAD // NSUB
GRIDPTS = (-2.0, -2.0 / 3.0, 2.0 / 3.0, 2.0)
INV_DENOM = 0.75
R = 200
NBR = N // R
NGRAPH = 64
NCLS = 16


def _fastkan(x, lng, lnb, wspl, wbt, bb):
    mu = jnp.mean(x, axis=1, keepdims=True)
    var = jnp.mean((x - mu) ** 2, axis=1, keepdims=True)
    xn = (x - mu) * lax.rsqrt(var + 1e-5) * lng + lnb
    h = jnp.dot(jax.nn.silu(x), wbt, preferred_element_type=jnp.float32) + bb
    for g in range(4):
        basis = jnp.exp(-(((xn - GRIDPTS[g]) * INV_DENOM) ** 2))
        h = h + jnp.dot(basis, wspl[g], preferred_element_type=jnp.float32)
    return h


def _att_out(acc4, bias):
    s = acc4[0] + acc4[1]
    den0 = s[0, :, OC:OC + 1] + 1e-16
    den1 = s[1, :, OC:OC + 1] + 1e-16
    x = jnp.concatenate([s[0, :, :OC] / den0, s[1, :, :OC] / den1],
                        axis=1) + bias
    return jax.nn.silu(x)


def _emit_ext(h, swa, swb, ha_ref, hb_ref, tab_ref):
    rows = h.shape[0]
    asc = jnp.dot(h, swa, preferred_element_type=jnp.float32)
    adt = jnp.dot(h, swb, preferred_element_type=jnp.float32)
    z1 = jnp.zeros((rows, 1), jnp.float32)
    z6 = jnp.zeros((rows, 6), jnp.float32)
    ha_ref[...] = jnp.concatenate([h[:, :OC], z1, asc[:, 0:1], z6], axis=1)
    hb_ref[...] = jnp.concatenate([h[:, OC:], z1, asc[:, 1:2], z6], axis=1)
    tab_ref[...] = jnp.concatenate([adt, z6], axis=1)


def _tc_in_body(x_ref, lng_ref, lnb_ref, wspl_ref, wbt_ref, bb_ref, swa_ref,
                swb_ref, ha_ref, hb_ref, tab_ref):
    h = _fastkan(x_ref[...], lng_ref[...], lnb_ref[...], wspl_ref[...],
                 wbt_ref[...], bb_ref[...])
    _emit_ext(h, swa_ref[...], swb_ref[...], ha_ref, hb_ref, tab_ref)


def _tc_mid_body(acc_ref, bias_ref, lng_ref, lnb_ref, wspl_ref, wbt_ref,
                 bb_ref, swa_ref, swb_ref, ha_ref, hb_ref, tab_ref):
    x = _att_out(acc_ref[...], bias_ref[...])
    h = _fastkan(x, lng_ref[...], lnb_ref[...], wspl_ref[...], wbt_ref[...],
                 bb_ref[...])
    _emit_ext(h, swa_ref[...], swb_ref[...], ha_ref, hb_ref, tab_ref)


def _tc_pool_body(acc_ref, bias_ref, batch_ref, pooled_ref):
    x = _att_out(acc_ref[...], bias_ref[...])
    b = batch_ref[...].reshape(1, R)
    ids = lax.broadcasted_iota(jnp.int32, (NGRAPH, R), 0)
    oh = jnp.where(ids == b, 1.0, 0.0)
    contrib = jnp.dot(oh, x, preferred_element_type=jnp.float32)

    @pl.when(pl.program_id(0) == 0)
    def _():
        pooled_ref[...] = contrib

    @pl.when(pl.program_id(0) != 0)
    def _():
        pooled_ref[...] = pooled_ref[...] + contrib


def _tc_head_body(p_ref, lng_ref, lnb_ref, wspl_ref, wbt_ref, bb_ref,
                  out_ref):
    h = _fastkan(p_ref[...], lng_ref[...], lnb_ref[...], wspl_ref[...],
                 wbt_ref[...], bb_ref[...])
    m = jnp.max(h, axis=1, keepdims=True)
    lse = jnp.log(jnp.sum(jnp.exp(h - m), axis=1, keepdims=True)) + m
    out_ref[...] = h - lse


def _full(shape):
    nd = len(shape)
    return pl.BlockSpec(shape, lambda j: (0,) * nd)


_ACC_SPEC = pl.BlockSpec((2, 2, R, PEXT), lambda j: (0, 0, j, 0))
_EXT_OUT = [
    pl.BlockSpec((R, PEXT), lambda j: (j, 0)),
    pl.BlockSpec((R, PEXT), lambda j: (j, 0)),
    pl.BlockSpec((R, 8), lambda j: (j, 0)),
]
_EXT_SHAPE = [
    jax.ShapeDtypeStruct((N, PEXT), jnp.float32),
    jax.ShapeDtypeStruct((N, PEXT), jnp.float32),
    jax.ShapeDtypeStruct((N, 8), jnp.float32),
]


def _tc_transform(x, lng, lnb, wspl, wbt, bb, swa, swb):
    return pl.pallas_call(
        _tc_in_body,
        grid=(NBR,),
        in_specs=[
            pl.BlockSpec((R, D), lambda j: (j, 0)),
            _full((1, D)), _full((1, D)), _full((4, D, D)),
            _full((D, D)), _full((1, D)), _full((D, 2)), _full((D, 2)),
        ],
        out_specs=_EXT_OUT,
        out_shape=_EXT_SHAPE,
    )(x, lng, lnb, wspl, wbt, bb, swa, swb)


def _tc_mid(accb, bias, lng, lnb, wspl, wbt, bb, swa, swb):
    return pl.pallas_call(
        _tc_mid_body,
        grid=(NBR,),
        in_specs=[
            _ACC_SPEC,
            _full((1, D)), _full((1, D)), _full((1, D)), _full((4, D, D)),
            _full((D, D)), _full((1, D)), _full((D, 2)), _full((D, 2)),
        ],
        out_specs=_EXT_OUT,
        out_shape=_EXT_SHAPE,
    )(accb, bias, lng, lnb, wspl, wbt, bb, swa, swb)


def _tc_pool(accb, bias, batch3):
    return pl.pallas_call(
        _tc_pool_body,
        grid=(NBR,),
        in_specs=[
            _ACC_SPEC,
            _full((1, D)),
            pl.BlockSpec((1, 1, R), lambda j: (j, 0, 0)),
        ],
        out_specs=pl.BlockSpec((NGRAPH, D), lambda j: (0, 0)),
        out_shape=jax.ShapeDtypeStruct((NGRAPH, D), jnp.float32),
    )(accb, bias, batch3)


def _tc_head(pooled, lng, lnb, wspl, wbt, bb):
    return pl.pallas_call(
        _tc_head_body,
        grid=(1,),
        in_specs=[
            _full((NGRAPH, D)), _full((1, D)), _full((1, D)),
            _full((4, D, NCLS)), _full((D, NCLS)), _full((1, NCLS)),
        ],
        out_specs=_full((NGRAPH, NCLS)),
        out_shape=jax.ShapeDtypeStruct((NGRAPH, NCLS), jnp.float32),
    )(pooled, lng, lnb, wspl, wbt, bb)


def _sc_body(ha, hb, tab, srcm, dstm, out,
             sidx_v, didx_v, rows_v, drows_v, wb_v, acc, sem0, sem1):
    c = lax.axis_index("c")
    s = lax.axis_index("s")
    wid = c * NSUB + s
    base = wid * EPW
    rslice = pl.ds(s * TROWS, TROWS)
    sems = (sem0, sem1)

    z16 = jnp.zeros((16,), jnp.int32)
    zf16 = jnp.zeros((16,), jnp.float32)
    lane = lax.iota(jnp.int32, 16)

    for p, href in enumerate((ha, hb)):

        def issue(j, b):
            pltpu.sync_copy(srcm.at[wid, pl.ds(j, 1)], sidx_v.at[b])
            pltpu.sync_copy(dstm.at[wid, pl.ds(j, 1)], didx_v.at[b])
            pltpu.async_copy(href.at[sidx_v.at[b, 0]], rows_v.at[b], sems[b])
            pltpu.async_copy(tab.at[didx_v.at[b, 0]], drows_v.at[b], sems[b])

        def wait_gathers(b):
            pltpu.make_async_copy(href.at[sidx_v.at[b, 0]], rows_v.at[b],
                                  sems[b]).wait()
            pltpu.make_async_copy(tab.at[didx_v.at[b, 0]], drows_v.at[b],
                                  sems[b]).wait()

        @plsc.parallel_loop(0, BLK, unroll=8)
        def _(i):
            for k in range(4):
                rows_v[0, i, pl.ds(k * 16, 16)] = zf16
            plsc.store_scatter(rows_v.at[0], [z16 + i, OC + lane], zf16,
                               mask=lane < PEXT - OC)

        for t in range(TROWS // BLK):
            pltpu.sync_copy(rows_v.at[0],
                            acc.at[pl.ds(s * TROWS + t * BLK, BLK)])
        issue(0, 0)
        issue(1, 1)
        plsc.subcore_barrier()

        def grp(jo, carry):
            for b in (0, 1):
                j = jo * 2 + b
                wait_gathers(b)
                for g in range(BLK // 16):
                    sl = pl.ds(g * 16, 16)
                    eidx = g * 16 + lane
                    a_s = plsc.load_gather(rows_v.at[b],
                                           [eidx, z16 + OC + 1])
                    a_d = plsc.load_gather(drows_v.at[b], [eidx, z16 + p])
                    e = a_s + a_d
                    e = jnp.where(e >= 0.0, e, 0.2 * e)
                    eid = base + j * BLK + eidx
                    wb_v[0, sl] = jnp.where(eid < E_TOT, jnp.exp(e), 0.0)

                @plsc.parallel_loop(0, BLK, unroll=8)
                def _(i):
                    ii = z16 + i
                    w = plsc.load_gather(wb_v, [z16, ii])
                    for k in range(4):
                        ksl = pl.ds(k * 16, 16)
                        rows_v[b, i, ksl] = rows_v[b, i, ksl] * w
                    tailv = jnp.where(lane == 0, w, 0.0)
                    plsc.store_scatter(rows_v.at[b], [ii, OC + lane], tailv,
                                       mask=lane < PEXT - OC)

                pltpu.sync_copy(rows_v.at[b], acc.at[didx_v.at[b, 0]],
                                add=True)
                jn = j + 2

                @pl.when(jn < NBLK)
                def _():
                    issue(jn, b)
            return carry

        lax.fori_loop(0, NBLK // 2, grp, 0)
        plsc.subcore_barrier()
        pltpu.sync_copy(acc.at[rslice], out.at[c, p, rslice])


_sc_aggregate = pl.kernel(
    _sc_body,
    out_type=jax.ShapeDtypeStruct((NCORE, 2, N_PAD, PEXT), jnp.float32),
    mesh=plsc.VectorSubcoreMesh(core_axis_name="c", subcore_axis_name="s"),
    compiler_params=pltpu.CompilerParams(
        use_tc_tiling_on_sc=False, needs_layout_passes=False),
    scratch_types=[
        pltpu.VMEM((2, 1, BLK), jnp.int32),
        pltpu.VMEM((2, 1, BLK), jnp.int32),
        pltpu.VMEM((2, BLK, PEXT), jnp.float32),
        pltpu.VMEM((2, BLK, 8), jnp.float32),
        pltpu.VMEM((1, BLK), jnp.float32),
        pltpu.VMEM_SHARED((N_PAD, PEXT), jnp.float32),
        pltpu.SemaphoreType.DMA,
        pltpu.SemaphoreType.DMA,
    ],
)


def _score_mats(asrc, adst):
    swa = jnp.zeros((D, 2), jnp.float32)
    swa = swa.at[:OC, 0].set(asrc[0])
    swa = swa.at[OC:, 1].set(asrc[1])
    swb = jnp.zeros((D, 2), jnp.float32)
    swb = swb.at[:OC, 0].set(adst[0])
    swb = swb.at[OC:, 1].set(adst[1])
    return swa, swb


def kernel(x, edge_index, batch, ln_g0, ln_b0, Ws0, Wb0, bb0, asrc0, adst0,
           bias0, ln_g1, ln_b1, Ws1, Wb1, bb1, asrc1, adst1, bias1,
           ln_gr, ln_br, Wsr, Wbr, bbr):
    loops = jnp.arange(N, dtype=edge_index.dtype)
    pad = jnp.arange(E_PAD - E_TOT, dtype=edge_index.dtype) % N
    src3 = jnp.concatenate([edge_index[0], loops, pad]).astype(jnp.int32)
    dst3 = jnp.concatenate([edge_index[1], loops, pad]).astype(jnp.int32)
    src3 = src3.reshape(NW, NBLK, BLK)
    dst3 = dst3.reshape(NW, NBLK, BLK)
    batch3 = batch.astype(jnp.int32).reshape(NBR, 1, R)

    def prep(Ws, out_ch):
        return Ws.reshape(out_ch, D, 4).transpose(2, 1, 0)

    lng0, lnb0 = ln_g0.reshape(1, D), ln_b0.reshape(1, D)
    lng1, lnb1 = ln_g1.reshape(1, D), ln_b1.reshape(1, D)
    lngr, lnbr = ln_gr.reshape(1, D), ln_br.reshape(1, D)
    swa0, swb0 = _score_mats(asrc0, adst0)
    swa1, swb1 = _score_mats(asrc1, adst1)

    ha, hb, tab = _tc_transform(x, lng0, lnb0, prep(Ws0, D), Wb0.T,
                                bb0.reshape(1, D), swa0, swb0)
    accb = _sc_aggregate(ha, hb, tab, src3, dst3)
    ha, hb, tab = _tc_mid(accb, bias0.reshape(1, D), lng1, lnb1,
                          prep(Ws1, D), Wb1.T, bb1.reshape(1, D),
                          swa1, swb1)
    accb = _sc_aggregate(ha, hb, tab, src3, dst3)
    pooled = _tc_pool(accb, bias1.reshape(1, D), batch3)
    return _tc_head(pooled, lngr, lnbr, prep(Wsr, NCLS), Wbr.T,
                    bbr.reshape(1, NCLS))


# speedup vs baseline: 3.2655x; 1.3764x over previous
import jax
import jax.numpy as jnp
from jax import lax
from jax.experimental import pallas as pl
from jax.experimental.pallas import tpu as pltpu
from jax.experimental.pallas import tpu_sc as plsc

N = 10000
E_RAW = 320000
E_TOT = E_RAW + N
D = 128
OC = 64
PEXT = 72
NCORE = 2
NSUB = 16
NW = NCORE * NSUB
BLK = 128
NBLK = 82
EPW = NBLK * BLK
E_PAD = NW * EPW
N_PAD = 10240
TROWS = N_PAD // NSUB
GRIDPTS = (-2.0, -2.0 / 3.0, 2.0 / 3.0, 2.0)
INV_DENOM = 0.75
R = 200
NBR = N // R
NGRAPH = 64
NCLS = 16


def _fastkan(x, lng, lnb, wspl, wbt, bb):
    mu = jnp.mean(x, axis=1, keepdims=True)
    var = jnp.mean((x - mu) ** 2, axis=1, keepdims=True)
    xn = (x - mu) * lax.rsqrt(var + 1e-5) * lng + lnb
    h = jnp.dot(jax.nn.silu(x), wbt, preferred_element_type=jnp.float32) + bb
    for g in range(4):
        basis = jnp.exp(-(((xn - GRIDPTS[g]) * INV_DENOM) ** 2))
        h = h + jnp.dot(basis, wspl[g], preferred_element_type=jnp.float32)
    return h


def _att_out(acc4, bias):
    s = acc4[0] + acc4[1]
    den0 = s[0, :, OC:OC + 1] + 1e-16
    den1 = s[1, :, OC:OC + 1] + 1e-16
    x = jnp.concatenate([s[0, :, :OC] / den0, s[1, :, :OC] / den1],
                        axis=1) + bias
    return jax.nn.silu(x)


def _emit_ext(h, swa, swb, ha_ref, hb_ref, tab_ref):
    rows = h.shape[0]
    asc = jnp.dot(h, swa, preferred_element_type=jnp.float32)
    adt = jnp.dot(h, swb, preferred_element_type=jnp.float32)
    z1 = jnp.zeros((rows, 1), jnp.float32)
    z6 = jnp.zeros((rows, 6), jnp.float32)
    ha_ref[...] = jnp.concatenate([h[:, :OC], z1, asc[:, 0:1], z6], axis=1)
    hb_ref[...] = jnp.concatenate([h[:, OC:], z1, asc[:, 1:2], z6], axis=1)
    tab_ref[...] = jnp.concatenate([adt, z6], axis=1)


def _tc_in_body(x_ref, lng_ref, lnb_ref, wspl_ref, wbt_ref, bb_ref, swa_ref,
                swb_ref, ha_ref, hb_ref, tab_ref):
    h = _fastkan(x_ref[...], lng_ref[...], lnb_ref[...], wspl_ref[...],
                 wbt_ref[...], bb_ref[...])
    _emit_ext(h, swa_ref[...], swb_ref[...], ha_ref, hb_ref, tab_ref)


def _tc_mid_body(acc_ref, bias_ref, lng_ref, lnb_ref, wspl_ref, wbt_ref,
                 bb_ref, swa_ref, swb_ref, ha_ref, hb_ref, tab_ref):
    x = _att_out(acc_ref[...], bias_ref[...])
    h = _fastkan(x, lng_ref[...], lnb_ref[...], wspl_ref[...], wbt_ref[...],
                 bb_ref[...])
    _emit_ext(h, swa_ref[...], swb_ref[...], ha_ref, hb_ref, tab_ref)


def _tc_pool_body(acc_ref, bias_ref, batch_ref, pooled_ref):
    x = _att_out(acc_ref[...], bias_ref[...])
    b = batch_ref[...].reshape(1, R)
    ids = lax.broadcasted_iota(jnp.int32, (NGRAPH, R), 0)
    oh = jnp.where(ids == b, 1.0, 0.0)
    contrib = jnp.dot(oh, x, preferred_element_type=jnp.float32)

    @pl.when(pl.program_id(0) == 0)
    def _():
        pooled_ref[...] = contrib

    @pl.when(pl.program_id(0) != 0)
    def _():
        pooled_ref[...] = pooled_ref[...] + contrib


def _tc_head_body(p_ref, lng_ref, lnb_ref, wspl_ref, wbt_ref, bb_ref,
                  out_ref):
    h = _fastkan(p_ref[...], lng_ref[...], lnb_ref[...], wspl_ref[...],
                 wbt_ref[...], bb_ref[...])
    m = jnp.max(h, axis=1, keepdims=True)
    lse = jnp.log(jnp.sum(jnp.exp(h - m), axis=1, keepdims=True)) + m
    out_ref[...] = h - lse


def _full(shape):
    nd = len(shape)
    return pl.BlockSpec(shape, lambda j: (0,) * nd)


_ACC_SPEC = pl.BlockSpec((2, 2, R, PEXT), lambda j: (0, 0, j, 0))
_EXT_OUT = [
    pl.BlockSpec((R, PEXT), lambda j: (j, 0)),
    pl.BlockSpec((R, PEXT), lambda j: (j, 0)),
    pl.BlockSpec((R, 8), lambda j: (j, 0)),
]
_EXT_SHAPE = [
    jax.ShapeDtypeStruct((N, PEXT), jnp.float32),
    jax.ShapeDtypeStruct((N, PEXT), jnp.float32),
    jax.ShapeDtypeStruct((N, 8), jnp.float32),
]


def _tc_transform(x, lng, lnb, wspl, wbt, bb, swa, swb):
    return pl.pallas_call(
        _tc_in_body,
        grid=(NBR,),
        in_specs=[
            pl.BlockSpec((R, D), lambda j: (j, 0)),
            _full((1, D)), _full((1, D)), _full((4, D, D)),
            _full((D, D)), _full((1, D)), _full((D, 2)), _full((D, 2)),
        ],
        out_specs=_EXT_OUT,
        out_shape=_EXT_SHAPE,
    )(x, lng, lnb, wspl, wbt, bb, swa, swb)


def _tc_mid(accb, bias, lng, lnb, wspl, wbt, bb, swa, swb):
    return pl.pallas_call(
        _tc_mid_body,
        grid=(NBR,),
        in_specs=[
            _ACC_SPEC,
            _full((1, D)), _full((1, D)), _full((1, D)), _full((4, D, D)),
            _full((D, D)), _full((1, D)), _full((D, 2)), _full((D, 2)),
        ],
        out_specs=_EXT_OUT,
        out_shape=_EXT_SHAPE,
    )(accb, bias, lng, lnb, wspl, wbt, bb, swa, swb)


def _tc_pool(accb, bias, batch3):
    return pl.pallas_call(
        _tc_pool_body,
        grid=(NBR,),
        in_specs=[
            _ACC_SPEC,
            _full((1, D)),
            pl.BlockSpec((1, 1, R), lambda j: (j, 0, 0)),
        ],
        out_specs=pl.BlockSpec((NGRAPH, D), lambda j: (0, 0)),
        out_shape=jax.ShapeDtypeStruct((NGRAPH, D), jnp.float32),
    )(accb, bias, batch3)


def _tc_head(pooled, lng, lnb, wspl, wbt, bb):
    return pl.pallas_call(
        _tc_head_body,
        grid=(1,),
        in_specs=[
            _full((NGRAPH, D)), _full((1, D)), _full((1, D)),
            _full((4, D, NCLS)), _full((D, NCLS)), _full((1, NCLS)),
        ],
        out_specs=_full((NGRAPH, NCLS)),
        out_shape=jax.ShapeDtypeStruct((NGRAPH, NCLS), jnp.float32),
    )(pooled, lng, lnb, wspl, wbt, bb)


def _sc_body(ha, hb, tab, em, out,
             packed_v, sidx_v, didx_v, rows_v, drows_v, wb_v, acc,
             sem0, sem1):
    c = lax.axis_index("c")
    s = lax.axis_index("s")
    wid = c * NSUB + s
    base = wid * EPW
    rslice = pl.ds(s * TROWS, TROWS)
    sems = (sem0, sem1)

    z16 = jnp.zeros((16,), jnp.int32)
    zf16 = jnp.zeros((16,), jnp.float32)
    lane = lax.iota(jnp.int32, 16)

    pltpu.sync_copy(em.at[wid], packed_v)

    for p, href in enumerate((ha, hb)):

        def issue(j, b):
            for g in range(BLK // 16):
                sl = pl.ds(g * 16, 16)
                pk = packed_v[j, sl]
                sidx_v[b, 0, sl] = jnp.bitwise_and(pk, 0xFFFF)
                didx_v[b, 0, sl] = lax.shift_right_logical(pk, 16)
            pltpu.async_copy(href.at[sidx_v.at[b, 0]], rows_v.at[b], sems[b])
            pltpu.async_copy(tab.at[didx_v.at[b, 0]], drows_v.at[b], sems[b])

        def wait_gathers(b):
            pltpu.make_async_copy(href.at[sidx_v.at[b, 0]], rows_v.at[b],
                                  sems[b]).wait()
            pltpu.make_async_copy(tab.at[didx_v.at[b, 0]], drows_v.at[b],
                                  sems[b]).wait()

        @plsc.parallel_loop(0, BLK, unroll=8)
        def _(i):
            for k in range(4):
                rows_v[0, i, pl.ds(k * 16, 16)] = zf16
            plsc.store_scatter(rows_v.at[0], [z16 + i, OC + lane], zf16,
                               mask=lane < PEXT - OC)

        for t in range(TROWS // BLK):
            pltpu.sync_copy(rows_v.at[0],
                            acc.at[pl.ds(s * TROWS + t * BLK, BLK)])
        issue(0, 0)
        issue(1, 1)
        plsc.subcore_barrier()

        def grp(jo, carry):
            for b in (0, 1):
                j = jo * 2 + b
                wait_gathers(b)
                for g in range(BLK // 16):
                    sl = pl.ds(g * 16, 16)
                    eidx = g * 16 + lane
                    a_s = plsc.load_gather(rows_v.at[b],
                                           [eidx, z16 + OC + 1])
                    a_d = plsc.load_gather(drows_v.at[b], [eidx, z16 + p])
                    e = a_s + a_d
                    e = jnp.where(e >= 0.0, e, 0.2 * e)
                    eid = base + j * BLK + eidx
                    wb_v[0, sl] = jnp.where(eid < E_TOT, jnp.exp(e), 0.0)

                @plsc.parallel_loop(0, BLK, unroll=8)
                def _(i):
                    ii = z16 + i
                    w = plsc.load_gather(wb_v, [z16, ii])
                    for k in range(4):
                        ksl = pl.ds(k * 16, 16)
                        rows_v[b, i, ksl] = rows_v[b, i, ksl] * w
                    tailv = jnp.where(lane == 0, w, 0.0)
                    plsc.store_scatter(rows_v.at[b], [ii, OC + lane], tailv,
                                       mask=lane < PEXT - OC)

                pltpu.sync_copy(rows_v.at[b], acc.at[didx_v.at[b, 0]],
                                add=True)
                jn = j + 2

                @pl.when(jn < NBLK)
                def _():
                    issue(jn, b)
            return carry

        lax.fori_loop(0, NBLK // 2, grp, 0)
        plsc.subcore_barrier()
        pltpu.sync_copy(acc.at[rslice], out.at[c, p, rslice])


_sc_aggregate = pl.kernel(
    _sc_body,
    out_type=jax.ShapeDtypeStruct((NCORE, 2, N_PAD, PEXT), jnp.float32),
    mesh=plsc.VectorSubcoreMesh(core_axis_name="c", subcore_axis_name="s"),
    compiler_params=pltpu.CompilerParams(
        use_tc_tiling_on_sc=False, needs_layout_passes=False),
    scratch_types=[
        pltpu.VMEM((NBLK, BLK), jnp.int32),
        pltpu.VMEM((2, 1, BLK), jnp.int32),
        pltpu.VMEM((2, 1, BLK), jnp.int32),
        pltpu.VMEM((2, BLK, PEXT), jnp.float32),
        pltpu.VMEM((2, BLK, 8), jnp.float32),
        pltpu.VMEM((1, BLK), jnp.float32),
        pltpu.VMEM_SHARED((N_PAD, PEXT), jnp.float32),
        pltpu.SemaphoreType.DMA,
        pltpu.SemaphoreType.DMA,
    ],
)


def _score_mats(asrc, adst):
    swa = jnp.zeros((D, 2), jnp.float32)
    swa = swa.at[:OC, 0].set(asrc[0])
    swa = swa.at[OC:, 1].set(asrc[1])
    swb = jnp.zeros((D, 2), jnp.float32)
    swb = swb.at[:OC, 0].set(adst[0])
    swb = swb.at[OC:, 1].set(adst[1])
    return swa, swb


def kernel(x, edge_index, batch, ln_g0, ln_b0, Ws0, Wb0, bb0, asrc0, adst0,
           bias0, ln_g1, ln_b1, Ws1, Wb1, bb1, asrc1, adst1, bias1,
           ln_gr, ln_br, Wsr, Wbr, bbr):
    loops = jnp.arange(N, dtype=edge_index.dtype)
    pad = jnp.arange(E_PAD - E_TOT, dtype=edge_index.dtype) % N
    src3 = jnp.concatenate([edge_index[0], loops, pad]).astype(jnp.int32)
    dst3 = jnp.concatenate([edge_index[1], loops, pad]).astype(jnp.int32)
    em = (src3 | (dst3 << 16)).reshape(NW, NBLK, BLK)
    batch3 = batch.astype(jnp.int32).reshape(NBR, 1, R)

    def prep(Ws, out_ch):
        return Ws.reshape(out_ch, D, 4).transpose(2, 1, 0)

    lng0, lnb0 = ln_g0.reshape(1, D), ln_b0.reshape(1, D)
    lng1, lnb1 = ln_g1.reshape(1, D), ln_b1.reshape(1, D)
    lngr, lnbr = ln_gr.reshape(1, D), ln_br.reshape(1, D)
    swa0, swb0 = _score_mats(asrc0, adst0)
    swa1, swb1 = _score_mats(asrc1, adst1)

    ha, hb, tab = _tc_transform(x, lng0, lnb0, prep(Ws0, D), Wb0.T,
                                bb0.reshape(1, D), swa0, swb0)
    accb = _sc_aggregate(ha, hb, tab, em)
    ha, hb, tab = _tc_mid(accb, bias0.reshape(1, D), lng1, lnb1,
                          prep(Ws1, D), Wb1.T, bb1.reshape(1, D),
                          swa1, swb1)
    accb = _sc_aggregate(ha, hb, tab, em)
    pooled = _tc_pool(accb, bias1.reshape(1, D), batch3)
    return _tc_head(pooled, lngr, lnbr, prep(Wsr, NCLS), Wbr.T,
                    bbr.reshape(1, NCLS))


# speedup vs baseline: 3.4269x; 1.0494x over previous
import jax
import jax.numpy as jnp
from jax import lax
from jax.experimental import pallas as pl
from jax.experimental.pallas import tpu as pltpu
from jax.experimental.pallas import tpu_sc as plsc

N = 10000
E_RAW = 320000
E_TOT = E_RAW + N
D = 128
OC = 64
PEXT = 72
NCORE = 2
NSUB = 16
NW = NCORE * NSUB
BLK = 128
NBLK = 82
EPW = NBLK * BLK
E_PAD = NW * EPW
N_PAD = 10240
TROWS = N_PAD // NSUB
GRIDPTS = (-2.0, -2.0 / 3.0, 2.0 / 3.0, 2.0)
INV_DENOM = 0.75
R = 400
NBR = N // R
NGRAPH = 64
NCLS = 16


def _fastkan(x, lng, lnb, wspl, wbt, bb):
    mu = jnp.mean(x, axis=1, keepdims=True)
    var = jnp.mean((x - mu) ** 2, axis=1, keepdims=True)
    xn = (x - mu) * lax.rsqrt(var + 1e-5) * lng + lnb
    h = jnp.dot(jax.nn.silu(x), wbt, preferred_element_type=jnp.float32) + bb
    for g in range(4):
        basis = jnp.exp(-(((xn - GRIDPTS[g]) * INV_DENOM) ** 2))
        h = h + jnp.dot(basis, wspl[g], preferred_element_type=jnp.float32)
    return h


def _att_out(acc4, bias):
    s = acc4[0] + acc4[1]
    den0 = s[0, :, OC:OC + 1] + 1e-16
    den1 = s[1, :, OC:OC + 1] + 1e-16
    x = jnp.concatenate([s[0, :, :OC] / den0, s[1, :, :OC] / den1],
                        axis=1) + bias
    return jax.nn.silu(x)


def _emit_ext(h, swa, swb, ha_ref, hb_ref, tab_ref):
    rows = h.shape[0]
    asc = jnp.dot(h, swa, preferred_element_type=jnp.float32)
    adt = jnp.dot(h, swb, preferred_element_type=jnp.float32)
    z1 = jnp.zeros((rows, 1), jnp.float32)
    z6 = jnp.zeros((rows, 6), jnp.float32)
    ha_ref[...] = jnp.concatenate([h[:, :OC], z1, asc[:, 0:1], z6], axis=1)
    hb_ref[...] = jnp.concatenate([h[:, OC:], z1, asc[:, 1:2], z6], axis=1)
    tab_ref[...] = jnp.concatenate([adt, z6], axis=1)


def _tc_in_body(x_ref, lng_ref, lnb_ref, wspl_ref, wbt_ref, bb_ref, swa_ref,
                swb_ref, ha_ref, hb_ref, tab_ref):
    h = _fastkan(x_ref[...], lng_ref[...], lnb_ref[...], wspl_ref[...],
                 wbt_ref[...], bb_ref[...])
    _emit_ext(h, swa_ref[...], swb_ref[...], ha_ref, hb_ref, tab_ref)


def _tc_mid_body(acc_ref, bias_ref, lng_ref, lnb_ref, wspl_ref, wbt_ref,
                 bb_ref, swa_ref, swb_ref, ha_ref, hb_ref, tab_ref):
    x = _att_out(acc_ref[...], bias_ref[...])
    h = _fastkan(x, lng_ref[...], lnb_ref[...], wspl_ref[...], wbt_ref[...],
                 bb_ref[...])
    _emit_ext(h, swa_ref[...], swb_ref[...], ha_ref, hb_ref, tab_ref)


def _tc_pool_body(acc_ref, bias_ref, batch_ref, lng_ref, lnb_ref,
                  wspl_ref, wbt_ref, bb_ref, out_ref, pooled_ref):
    x = _att_out(acc_ref[...], bias_ref[...])
    b = batch_ref[...].reshape(1, R)
    ids = lax.broadcasted_iota(jnp.int32, (NGRAPH, R), 0)
    oh = jnp.where(ids == b, 1.0, 0.0)
    contrib = jnp.dot(oh, x, preferred_element_type=jnp.float32)

    @pl.when(pl.program_id(0) == 0)
    def _():
        pooled_ref[...] = contrib

    @pl.when(pl.program_id(0) != 0)
    def _():
        pooled_ref[...] = pooled_ref[...] + contrib

    @pl.when(pl.program_id(0) == NBR - 1)
    def _():
        h = _fastkan(pooled_ref[...], lng_ref[...], lnb_ref[...],
                     wspl_ref[...], wbt_ref[...], bb_ref[...])
        m = jnp.max(h, axis=1, keepdims=True)
        lse = jnp.log(jnp.sum(jnp.exp(h - m), axis=1, keepdims=True)) + m
        out_ref[...] = h - lse


def _full(shape):
    nd = len(shape)
    return pl.BlockSpec(shape, lambda j: (0,) * nd)


_ACC_SPEC = pl.BlockSpec((2, 2, R, PEXT), lambda j: (0, 0, j, 0))
_EXT_OUT = [
    pl.BlockSpec((R, PEXT), lambda j: (j, 0)),
    pl.BlockSpec((R, PEXT), lambda j: (j, 0)),
    pl.BlockSpec((R, 8), lambda j: (j, 0)),
]
_EXT_SHAPE = [
    jax.ShapeDtypeStruct((N, PEXT), jnp.float32),
    jax.ShapeDtypeStruct((N, PEXT), jnp.float32),
    jax.ShapeDtypeStruct((N, 8), jnp.float32),
]


def _tc_transform(x, lng, lnb, wspl, wbt, bb, swa, swb):
    return pl.pallas_call(
        _tc_in_body,
        grid=(NBR,),
        in_specs=[
            pl.BlockSpec((R, D), lambda j: (j, 0)),
            _full((1, D)), _full((1, D)), _full((4, D, D)),
            _full((D, D)), _full((1, D)), _full((D, 2)), _full((D, 2)),
        ],
        out_specs=_EXT_OUT,
        out_shape=_EXT_SHAPE,
    )(x, lng, lnb, wspl, wbt, bb, swa, swb)


def _tc_mid(accb, bias, lng, lnb, wspl, wbt, bb, swa, swb):
    return pl.pallas_call(
        _tc_mid_body,
        grid=(NBR,),
        in_specs=[
            _ACC_SPEC,
            _full((1, D)), _full((1, D)), _full((1, D)), _full((4, D, D)),
            _full((D, D)), _full((1, D)), _full((D, 2)), _full((D, 2)),
        ],
        out_specs=_EXT_OUT,
        out_shape=_EXT_SHAPE,
    )(accb, bias, lng, lnb, wspl, wbt, bb, swa, swb)


def _tc_pool(accb, bias, batch3, lng, lnb, wspl, wbt, bb):
    return pl.pallas_call(
        _tc_pool_body,
        grid=(NBR,),
        in_specs=[
            _ACC_SPEC,
            _full((1, D)),
            pl.BlockSpec((1, 1, R), lambda j: (j, 0, 0)),
            _full((1, D)), _full((1, D)), _full((4, D, NCLS)),
            _full((D, NCLS)), _full((1, NCLS)),
        ],
        out_specs=_full((NGRAPH, NCLS)),
        out_shape=jax.ShapeDtypeStruct((NGRAPH, NCLS), jnp.float32),
        scratch_shapes=[pltpu.VMEM((NGRAPH, D), jnp.float32)],
    )(accb, bias, batch3, lng, lnb, wspl, wbt, bb)


def _sc_body(ha, hb, tab, em, out,
             packed_v, sidx_v, didx_v, rows_v, drows_v, wb_v, acc,
             sem0, sem1):
    c = lax.axis_index("c")
    s = lax.axis_index("s")
    wid = c * NSUB + s
    base = wid * EPW
    rslice = pl.ds(s * TROWS, TROWS)
    sems = (sem0, sem1)

    z16 = jnp.zeros((16,), jnp.int32)
    zf16 = jnp.zeros((16,), jnp.float32)
    lane = lax.iota(jnp.int32, 16)

    pltpu.sync_copy(em.at[wid], packed_v)

    for p, href in enumerate((ha, hb)):

        def issue(j, b):
            for g in range(BLK // 16):
                sl = pl.ds(g * 16, 16)
                pk = packed_v[j, sl]
                sidx_v[b, 0, sl] = jnp.bitwise_and(pk, 0xFFFF)
                didx_v[b, 0, sl] = lax.shift_right_logical(pk, 16)
            pltpu.async_copy(href.at[sidx_v.at[b, 0]], rows_v.at[b], sems[b])
            pltpu.async_copy(tab.at[didx_v.at[b, 0]], drows_v.at[b], sems[b])

        def wait_gathers(b):
            pltpu.make_async_copy(href.at[sidx_v.at[b, 0]], rows_v.at[b],
                                  sems[b]).wait()
            pltpu.make_async_copy(tab.at[didx_v.at[b, 0]], drows_v.at[b],
                                  sems[b]).wait()

        @plsc.parallel_loop(0, BLK, unroll=8)
        def _(i):
            for k in range(4):
                rows_v[0, i, pl.ds(k * 16, 16)] = zf16
            plsc.store_scatter(rows_v.at[0], [z16 + i, OC + lane], zf16,
                               mask=lane < PEXT - OC)

        for t in range(TROWS // BLK):
            pltpu.sync_copy(rows_v.at[0],
                            acc.at[pl.ds(s * TROWS + t * BLK, BLK)])
        issue(0, 0)
        issue(1, 1)
        plsc.subcore_barrier()

        def grp(jo, carry):
            for b in (0, 1):
                j = jo * 2 + b
                wait_gathers(b)
                for g in range(BLK // 16):
                    sl = pl.ds(g * 16, 16)
                    eidx = g * 16 + lane
                    a_s = plsc.load_gather(rows_v.at[b],
                                           [eidx, z16 + OC + 1])
                    a_d = plsc.load_gather(drows_v.at[b], [eidx, z16 + p])
                    e = a_s + a_d
                    e = jnp.where(e >= 0.0, e, 0.2 * e)
                    eid = base + j * BLK + eidx
                    wb_v[0, sl] = jnp.where(eid < E_TOT, jnp.exp(e), 0.0)

                @plsc.parallel_loop(0, BLK, unroll=8)
                def _(i):
                    ii = z16 + i
                    w = plsc.load_gather(wb_v, [z16, ii])
                    for k in range(4):
                        ksl = pl.ds(k * 16, 16)
                        rows_v[b, i, ksl] = rows_v[b, i, ksl] * w
                    tailv = jnp.where(lane == 0, w, 0.0)
                    plsc.store_scatter(rows_v.at[b], [ii, OC + lane], tailv,
                                       mask=lane < PEXT - OC)

                pltpu.sync_copy(rows_v.at[b], acc.at[didx_v.at[b, 0]],
                                add=True)
                jn = j + 2

                @pl.when(jn < NBLK)
                def _():
                    issue(jn, b)
            return carry

        lax.fori_loop(0, NBLK // 2, grp, 0)
        plsc.subcore_barrier()
        pltpu.sync_copy(acc.at[rslice], out.at[c, p, rslice])


_sc_aggregate = pl.kernel(
    _sc_body,
    out_type=jax.ShapeDtypeStruct((NCORE, 2, N_PAD, PEXT), jnp.float32),
    mesh=plsc.VectorSubcoreMesh(core_axis_name="c", subcore_axis_name="s"),
    compiler_params=pltpu.CompilerParams(
        use_tc_tiling_on_sc=False, needs_layout_passes=False),
    scratch_types=[
        pltpu.VMEM((NBLK, BLK), jnp.int32),
        pltpu.VMEM((2, 1, BLK), jnp.int32),
        pltpu.VMEM((2, 1, BLK), jnp.int32),
        pltpu.VMEM((2, BLK, PEXT), jnp.float32),
        pltpu.VMEM((2, BLK, 8), jnp.float32),
        pltpu.VMEM((1, BLK), jnp.float32),
        pltpu.VMEM_SHARED((N_PAD, PEXT), jnp.float32),
        pltpu.SemaphoreType.DMA,
        pltpu.SemaphoreType.DMA,
    ],
)


def _score_mats(asrc, adst):
    swa = jnp.zeros((D, 2), jnp.float32)
    swa = swa.at[:OC, 0].set(asrc[0])
    swa = swa.at[OC:, 1].set(asrc[1])
    swb = jnp.zeros((D, 2), jnp.float32)
    swb = swb.at[:OC, 0].set(adst[0])
    swb = swb.at[OC:, 1].set(adst[1])
    return swa, swb


def kernel(x, edge_index, batch, ln_g0, ln_b0, Ws0, Wb0, bb0, asrc0, adst0,
           bias0, ln_g1, ln_b1, Ws1, Wb1, bb1, asrc1, adst1, bias1,
           ln_gr, ln_br, Wsr, Wbr, bbr):
    loops = jnp.arange(N, dtype=edge_index.dtype)
    pad = jnp.arange(E_PAD - E_TOT, dtype=edge_index.dtype) % N
    src3 = jnp.concatenate([edge_index[0], loops, pad]).astype(jnp.int32)
    dst3 = jnp.concatenate([edge_index[1], loops, pad]).astype(jnp.int32)
    em = (src3 | (dst3 << 16)).reshape(NW, NBLK, BLK)
    batch3 = batch.astype(jnp.int32).reshape(NBR, 1, R)

    def prep(Ws, out_ch):
        return Ws.reshape(out_ch, D, 4).transpose(2, 1, 0)

    lng0, lnb0 = ln_g0.reshape(1, D), ln_b0.reshape(1, D)
    lng1, lnb1 = ln_g1.reshape(1, D), ln_b1.reshape(1, D)
    lngr, lnbr = ln_gr.reshape(1, D), ln_br.reshape(1, D)
    swa0, swb0 = _score_mats(asrc0, adst0)
    swa1, swb1 = _score_mats(asrc1, adst1)

    ha, hb, tab = _tc_transform(x, lng0, lnb0, prep(Ws0, D), Wb0.T,
                                bb0.reshape(1, D), swa0, swb0)
    accb = _sc_aggregate(ha, hb, tab, em)
    ha, hb, tab = _tc_mid(accb, bias0.reshape(1, D), lng1, lnb1,
                          prep(Ws1, D), Wb1.T, bb1.reshape(1, D),
                          swa1, swb1)
    accb = _sc_aggregate(ha, hb, tab, em)
    return _tc_pool(accb, bias1.reshape(1, D), batch3, lngr, lnbr,
                    prep(Wsr, NCLS), Wbr.T, bbr.reshape(1, NCLS))


# speedup vs baseline: 3.5663x; 1.0407x over previous
import jax
import jax.numpy as jnp
from jax import lax
from jax.experimental import pallas as pl
from jax.experimental.pallas import tpu as pltpu
from jax.experimental.pallas import tpu_sc as plsc

N = 10000
E_RAW = 320000
E_TOT = E_RAW + N
D = 128
OC = 64
PEXT = 72
NCORE = 2
NSUB = 16
NW = NCORE * NSUB
BLK = 128
NBLK = 82
EPW = NBLK * BLK
E_PAD = NW * EPW
N_PAD = 10240
TROWS = N_PAD // NSUB
GRIDPTS = (-2.0, -2.0 / 3.0, 2.0 / 3.0, 2.0)
INV_DENOM = 0.75
R = 1000
NBR = N // R
NGRAPH = 64
NCLS = 16


def _fastkan(x, lng, lnb, wspl, wbt, bb):
    mu = jnp.mean(x, axis=1, keepdims=True)
    var = jnp.mean((x - mu) ** 2, axis=1, keepdims=True)
    xn = (x - mu) * lax.rsqrt(var + 1e-5) * lng + lnb
    h = jnp.dot(jax.nn.silu(x), wbt, preferred_element_type=jnp.float32) + bb
    for g in range(4):
        basis = jnp.exp(-(((xn - GRIDPTS[g]) * INV_DENOM) ** 2))
        h = h + jnp.dot(basis, wspl[g], preferred_element_type=jnp.float32)
    return h


def _att_out(acc4, bias):
    s = acc4[0] + acc4[1]
    den0 = s[0, :, OC:OC + 1] + 1e-16
    den1 = s[1, :, OC:OC + 1] + 1e-16
    x = jnp.concatenate([s[0, :, :OC] / den0, s[1, :, :OC] / den1],
                        axis=1) + bias
    return jax.nn.silu(x)


def _emit_ext(h, swa, swb, ha_ref, hb_ref, tab_ref):
    rows = h.shape[0]
    asc = jnp.dot(h, swa, preferred_element_type=jnp.float32)
    adt = jnp.dot(h, swb, preferred_element_type=jnp.float32)
    z1 = jnp.zeros((rows, 1), jnp.float32)
    z6 = jnp.zeros((rows, 6), jnp.float32)
    ha_ref[...] = jnp.concatenate([h[:, :OC], z1, asc[:, 0:1], z6], axis=1)
    hb_ref[...] = jnp.concatenate([h[:, OC:], z1, asc[:, 1:2], z6], axis=1)
    tab_ref[...] = jnp.concatenate([adt, z6], axis=1)


def _tc_in_body(x_ref, lng_ref, lnb_ref, wspl_ref, wbt_ref, bb_ref, swa_ref,
                swb_ref, ha_ref, hb_ref, tab_ref):
    h = _fastkan(x_ref[...], lng_ref[...], lnb_ref[...], wspl_ref[...],
                 wbt_ref[...], bb_ref[...])
    _emit_ext(h, swa_ref[...], swb_ref[...], ha_ref, hb_ref, tab_ref)


def _tc_mid_body(acc_ref, bias_ref, lng_ref, lnb_ref, wspl_ref, wbt_ref,
                 bb_ref, swa_ref, swb_ref, ha_ref, hb_ref, tab_ref):
    x = _att_out(acc_ref[...], bias_ref[...])
    h = _fastkan(x, lng_ref[...], lnb_ref[...], wspl_ref[...], wbt_ref[...],
                 bb_ref[...])
    _emit_ext(h, swa_ref[...], swb_ref[...], ha_ref, hb_ref, tab_ref)


def _tc_pool_body(acc_ref, bias_ref, batch_ref, lng_ref, lnb_ref,
                  wspl_ref, wbt_ref, bb_ref, out_ref, pooled_ref):
    x = _att_out(acc_ref[...], bias_ref[...])
    b = batch_ref[...].reshape(1, R)
    ids = lax.broadcasted_iota(jnp.int32, (NGRAPH, R), 0)
    oh = jnp.where(ids == b, 1.0, 0.0)
    contrib = jnp.dot(oh, x, preferred_element_type=jnp.float32)

    @pl.when(pl.program_id(0) == 0)
    def _():
        pooled_ref[...] = contrib

    @pl.when(pl.program_id(0) != 0)
    def _():
        pooled_ref[...] = pooled_ref[...] + contrib

    @pl.when(pl.program_id(0) == NBR - 1)
    def _():
        h = _fastkan(pooled_ref[...], lng_ref[...], lnb_ref[...],
                     wspl_ref[...], wbt_ref[...], bb_ref[...])
        m = jnp.max(h, axis=1, keepdims=True)
        lse = jnp.log(jnp.sum(jnp.exp(h - m), axis=1, keepdims=True)) + m
        out_ref[...] = h - lse


def _full(shape):
    nd = len(shape)
    return pl.BlockSpec(shape, lambda j: (0,) * nd)


_ACC_SPEC = pl.BlockSpec((2, 2, R, PEXT), lambda j: (0, 0, j, 0))
_EXT_OUT = [
    pl.BlockSpec((R, PEXT), lambda j: (j, 0)),
    pl.BlockSpec((R, PEXT), lambda j: (j, 0)),
    pl.BlockSpec((R, 8), lambda j: (j, 0)),
]
_EXT_SHAPE = [
    jax.ShapeDtypeStruct((N, PEXT), jnp.float32),
    jax.ShapeDtypeStruct((N, PEXT), jnp.float32),
    jax.ShapeDtypeStruct((N, 8), jnp.float32),
]


def _tc_transform(x, lng, lnb, wspl, wbt, bb, swa, swb):
    return pl.pallas_call(
        _tc_in_body,
        grid=(NBR,),
        in_specs=[
            pl.BlockSpec((R, D), lambda j: (j, 0)),
            _full((1, D)), _full((1, D)), _full((4, D, D)),
            _full((D, D)), _full((1, D)), _full((D, 2)), _full((D, 2)),
        ],
        out_specs=_EXT_OUT,
        out_shape=_EXT_SHAPE,
    )(x, lng, lnb, wspl, wbt, bb, swa, swb)


def _tc_mid(accb, bias, lng, lnb, wspl, wbt, bb, swa, swb):
    return pl.pallas_call(
        _tc_mid_body,
        grid=(NBR,),
        in_specs=[
            _ACC_SPEC,
            _full((1, D)), _full((1, D)), _full((1, D)), _full((4, D, D)),
            _full((D, D)), _full((1, D)), _full((D, 2)), _full((D, 2)),
        ],
        out_specs=_EXT_OUT,
        out_shape=_EXT_SHAPE,
    )(accb, bias, lng, lnb, wspl, wbt, bb, swa, swb)


def _tc_pool(accb, bias, batch3, lng, lnb, wspl, wbt, bb):
    return pl.pallas_call(
        _tc_pool_body,
        grid=(NBR,),
        in_specs=[
            _ACC_SPEC,
            _full((1, D)),
            pl.BlockSpec((1, 1, R), lambda j: (j, 0, 0)),
            _full((1, D)), _full((1, D)), _full((4, D, NCLS)),
            _full((D, NCLS)), _full((1, NCLS)),
        ],
        out_specs=_full((NGRAPH, NCLS)),
        out_shape=jax.ShapeDtypeStruct((NGRAPH, NCLS), jnp.float32),
        scratch_shapes=[pltpu.VMEM((NGRAPH, D), jnp.float32)],
    )(accb, bias, batch3, lng, lnb, wspl, wbt, bb)


def _sc_body(ha, hb, tab, em, out,
             packed_v, sidx_v, didx_v, rows_v, drows_v, wb_v, acc,
             sem0, sem1):
    c = lax.axis_index("c")
    s = lax.axis_index("s")
    wid = c * NSUB + s
    base = wid * EPW
    rslice = pl.ds(s * TROWS, TROWS)
    sems = (sem0, sem1)

    z16 = jnp.zeros((16,), jnp.int32)
    zf16 = jnp.zeros((16,), jnp.float32)
    lane = lax.iota(jnp.int32, 16)

    pltpu.sync_copy(em.at[wid], packed_v)

    for p, href in enumerate((ha, hb)):

        def issue(j, b):
            for g in range(BLK // 16):
                sl = pl.ds(g * 16, 16)
                pk = packed_v[j, sl]
                sidx_v[b, 0, sl] = jnp.bitwise_and(pk, 0xFFFF)
                didx_v[b, 0, sl] = lax.shift_right_logical(pk, 16)
            pltpu.async_copy(href.at[sidx_v.at[b, 0]], rows_v.at[b], sems[b])
            pltpu.async_copy(tab.at[didx_v.at[b, 0]], drows_v.at[b], sems[b])

        def wait_gathers(b):
            pltpu.make_async_copy(href.at[sidx_v.at[b, 0]], rows_v.at[b],
                                  sems[b]).wait()
            pltpu.make_async_copy(tab.at[didx_v.at[b, 0]], drows_v.at[b],
                                  sems[b]).wait()

        @plsc.parallel_loop(0, BLK, unroll=8)
        def _(i):
            for k in range(4):
                rows_v[0, i, pl.ds(k * 16, 16)] = zf16
            plsc.store_scatter(rows_v.at[0], [z16 + i, OC + lane], zf16,
                               mask=lane < PEXT - OC)

        for t in range(TROWS // BLK):
            pltpu.sync_copy(rows_v.at[0],
                            acc.at[pl.ds(s * TROWS + t * BLK, BLK)])
        issue(0, 0)
        issue(1, 1)
        plsc.subcore_barrier()

        def grp(jo, carry):
            for b in (0, 1):
                j = jo * 2 + b
                wait_gathers(b)
                for g in range(BLK // 16):
                    sl = pl.ds(g * 16, 16)
                    eidx = g * 16 + lane
                    a_s = plsc.load_gather(rows_v.at[b],
                                           [eidx, z16 + OC + 1])
                    a_d = plsc.load_gather(drows_v.at[b], [eidx, z16 + p])
                    e = a_s + a_d
                    e = jnp.where(e >= 0.0, e, 0.2 * e)
                    eid = base + j * BLK + eidx
                    wb_v[0, sl] = jnp.where(eid < E_TOT, jnp.exp(e), 0.0)

                @plsc.parallel_loop(0, BLK, unroll=8)
                def _(i):
                    ii = z16 + i
                    w = plsc.load_gather(wb_v, [z16, ii])
                    for k in range(4):
                        ksl = pl.ds(k * 16, 16)
                        rows_v[b, i, ksl] = rows_v[b, i, ksl] * w
                    tailv = jnp.where(lane == 0, w, 0.0)
                    plsc.store_scatter(rows_v.at[b], [ii, OC + lane], tailv,
                                       mask=lane < PEXT - OC)

                pltpu.sync_copy(rows_v.at[b], acc.at[didx_v.at[b, 0]],
                                add=True)
                jn = j + 2

                @pl.when(jn < NBLK)
                def _():
                    issue(jn, b)
            return carry

        lax.fori_loop(0, NBLK // 2, grp, 0)
        plsc.subcore_barrier()
        pltpu.sync_copy(acc.at[rslice], out.at[c, p, rslice])


_sc_aggregate = pl.kernel(
    _sc_body,
    out_type=jax.ShapeDtypeStruct((NCORE, 2, N_PAD, PEXT), jnp.float32),
    mesh=plsc.VectorSubcoreMesh(core_axis_name="c", subcore_axis_name="s"),
    compiler_params=pltpu.CompilerParams(
        use_tc_tiling_on_sc=False, needs_layout_passes=False),
    scratch_types=[
        pltpu.VMEM((NBLK, BLK), jnp.int32),
        pltpu.VMEM((2, 1, BLK), jnp.int32),
        pltpu.VMEM((2, 1, BLK), jnp.int32),
        pltpu.VMEM((2, BLK, PEXT), jnp.float32),
        pltpu.VMEM((2, BLK, 8), jnp.float32),
        pltpu.VMEM((1, BLK), jnp.float32),
        pltpu.VMEM_SHARED((N_PAD, PEXT), jnp.float32),
        pltpu.SemaphoreType.DMA,
        pltpu.SemaphoreType.DMA,
    ],
)


def _score_mats(asrc, adst):
    swa = jnp.zeros((D, 2), jnp.float32)
    swa = swa.at[:OC, 0].set(asrc[0])
    swa = swa.at[OC:, 1].set(asrc[1])
    swb = jnp.zeros((D, 2), jnp.float32)
    swb = swb.at[:OC, 0].set(adst[0])
    swb = swb.at[OC:, 1].set(adst[1])
    return swa, swb


def kernel(x, edge_index, batch, ln_g0, ln_b0, Ws0, Wb0, bb0, asrc0, adst0,
           bias0, ln_g1, ln_b1, Ws1, Wb1, bb1, asrc1, adst1, bias1,
           ln_gr, ln_br, Wsr, Wbr, bbr):
    loops = jnp.arange(N, dtype=edge_index.dtype)
    pad = jnp.arange(E_PAD - E_TOT, dtype=edge_index.dtype) % N
    src3 = jnp.concatenate([edge_index[0], loops, pad]).astype(jnp.int32)
    dst3 = jnp.concatenate([edge_index[1], loops, pad]).astype(jnp.int32)
    em = (src3 | (dst3 << 16)).reshape(NW, NBLK, BLK)
    batch3 = batch.astype(jnp.int32).reshape(NBR, 1, R)

    def prep(Ws, out_ch):
        return Ws.reshape(out_ch, D, 4).transpose(2, 1, 0)

    lng0, lnb0 = ln_g0.reshape(1, D), ln_b0.reshape(1, D)
    lng1, lnb1 = ln_g1.reshape(1, D), ln_b1.reshape(1, D)
    lngr, lnbr = ln_gr.reshape(1, D), ln_br.reshape(1, D)
    swa0, swb0 = _score_mats(asrc0, adst0)
    swa1, swb1 = _score_mats(asrc1, adst1)

    ha, hb, tab = _tc_transform(x, lng0, lnb0, prep(Ws0, D), Wb0.T,
                                bb0.reshape(1, D), swa0, swb0)
    accb = _sc_aggregate(ha, hb, tab, em)
    ha, hb, tab = _tc_mid(accb, bias0.reshape(1, D), lng1, lnb1,
                          prep(Ws1, D), Wb1.T, bb1.reshape(1, D),
                          swa1, swb1)
    accb = _sc_aggregate(ha, hb, tab, em)
    return _tc_pool(accb, bias1.reshape(1, D), batch3, lngr, lnbr,
                    prep(Wsr, NCLS), Wbr.T, bbr.reshape(1, NCLS))
